# Initial kernel scaffold; baseline (speedup 1.0000x reference)
import jax
import jax.numpy as jnp
from jax.experimental import pallas as pl


def kernel(src_id, dst_id, rel_type, norm, weight, w_comp):
    raise NotImplementedError("write your pallas kernel here")


# speedup vs baseline: 4.8459x; 4.8459x over previous
import functools

import jax
import jax.numpy as jnp
from jax import lax
from jax.experimental import pallas as pl
from jax.experimental.pallas import tpu as pltpu
from jax.experimental.pallas import tpu_sc as plsc

NUM_NODES = 10000
N_EDGES = 320000
IN_FEAT = 10000
OUT_FEAT = 128
NUM_RELS = 8
NUM_BASES = 4

NC, NS, L = 2, 16, 16
NW = NC * NS
EPW = N_EDGES // NW
C = 80
NCH = EPW // C
RPT = NUM_NODES // NS
FV = OUT_FEAT // L


_QB = 400


def _embed_body(wc_ref, w3_ref, out_ref):
    for r in range(NUM_RELS):
        acc = wc_ref[r, 0] * w3_ref[:, 0, :]
        for b in range(1, NUM_BASES):
            acc = acc + wc_ref[r, b] * w3_ref[:, b, :]
        out_ref[r, :, :] = acc


def _build_embed(w_comp, w3):
    return pl.pallas_call(
        _embed_body,
        grid=(IN_FEAT // _QB,),
        in_specs=[
            pl.BlockSpec(memory_space=pltpu.SMEM),
            pl.BlockSpec((_QB, NUM_BASES, OUT_FEAT), lambda i: (i, 0, 0)),
        ],
        out_specs=pl.BlockSpec((NUM_RELS, _QB, OUT_FEAT), lambda i: (0, i, 0)),
        out_shape=jax.ShapeDtypeStruct((NUM_RELS, IN_FEAT, OUT_FEAT), jnp.float32),
    )(w_comp, w3)


def _gidx_body(src_ref, rel_ref, o_ref):
    j = rel_ref[...] * IN_FEAT + src_ref[...]
    o_ref[...] = (j & 7) * IN_FEAT + (j >> 3)


def _build_gidx(src2d, rel2d):
    return pl.pallas_call(
        _gidx_body,
        out_shape=jax.ShapeDtypeStruct(src2d.shape, jnp.int32),
    )(src2d, rel2d)


def _sc_body(emb, gidx2, dst2, norm2, out,
             idx_v, dst_v, norm_v, rows_v, h_sh, sem):
    c = lax.axis_index("c")
    s = lax.axis_index("s")
    wid = s * NC + c
    base = wid * NCH

    pltpu.sync_copy(gidx2.at[pl.ds(base, NCH)], idx_v)
    pltpu.sync_copy(dst2.at[pl.ds(base, NCH)], dst_v)
    pltpu.sync_copy(norm2.at[pl.ds(base, NCH)], norm_v)

    zero = jnp.zeros((L,), jnp.float32)

    def _zb(i, _):
        for k in range(FV):
            rows_v[i, pl.ds(k * L, L)] = zero
        return 0

    lax.fori_loop(0, C, _zb, 0)
    for t in range(8):
        off = s * RPT + t * C
        n = C if t < 7 else RPT - 7 * C
        pltpu.sync_copy(rows_v.at[pl.ds(0, n)], h_sh.at[pl.ds(off, n)])
    plsc.subcore_barrier()

    def _chunk(j, _):
        pltpu.async_copy(emb.at[idx_v.at[j]], rows_v, sem).wait()

        def _scale(g, _):
            nvec = norm_v[j, pl.ds(g * L, L)]
            for t in range(L):
                nv = nvec[t]
                i = g * L + t
                for k in range(FV):
                    sl = pl.ds(k * L, L)
                    rows_v[i, sl] = rows_v[i, sl] * nv
            return 0

        lax.fori_loop(0, C // L, _scale, 0)
        pltpu.sync_copy(rows_v, h_sh.at[dst_v.at[j]], add=True)
        return 0

    lax.fori_loop(0, NCH, _chunk, 0)
    plsc.subcore_barrier()

    for t in range(8):
        off = s * RPT + t * C
        n = C if t < 7 else RPT - 7 * C
        pltpu.sync_copy(h_sh.at[pl.ds(off, n)], rows_v.at[pl.ds(0, n)])
        pltpu.sync_copy(rows_v.at[pl.ds(0, n)], out.at[c, pl.ds(off, n)])


def _sc_call(emb, gidx2, dst2, norm2):
    mesh = plsc.VectorSubcoreMesh(core_axis_name="c", subcore_axis_name="s")
    f = pl.kernel(
        _sc_body,
        out_type=jax.ShapeDtypeStruct((NC, NUM_NODES, OUT_FEAT), jnp.float32),
        mesh=mesh,
        compiler_params=pltpu.CompilerParams(use_tc_tiling_on_sc=False),
        scratch_types=[
            pltpu.VMEM((NCH, C), jnp.int32),
            pltpu.VMEM((NCH, C), jnp.int32),
            pltpu.VMEM((NCH, C), jnp.float32),
            pltpu.VMEM((C, OUT_FEAT), jnp.float32),
            pltpu.VMEM_SHARED((NUM_NODES, OUT_FEAT), jnp.float32),
            pltpu.SemaphoreType.DMA,
        ],
    )
    return f(emb, gidx2, dst2, norm2)


_RB = 2000


def _sum_body(p_ref, o_ref):
    o_ref[...] = p_ref[0] + p_ref[1]


def _sum_partials(partial):
    return pl.pallas_call(
        _sum_body,
        grid=(NUM_NODES // _RB,),
        in_specs=[pl.BlockSpec((NC, _RB, OUT_FEAT), lambda i: (0, i, 0))],
        out_specs=pl.BlockSpec((_RB, OUT_FEAT), lambda i: (i, 0)),
        out_shape=jax.ShapeDtypeStruct((NUM_NODES, OUT_FEAT), jnp.float32),
    )(partial)


def kernel(src_id, dst_id, rel_type, norm, weight, w_comp):
    w3 = weight.reshape(IN_FEAT, NUM_BASES, OUT_FEAT)
    emb = _build_embed(w_comp, w3).reshape(NUM_RELS * IN_FEAT, OUT_FEAT)
    gidx = _build_gidx(src_id.reshape(-1, OUT_FEAT), rel_type.reshape(-1, OUT_FEAT))
    gidx2 = gidx.reshape(-1, C)
    dst2 = dst_id.reshape(-1, C)
    norm2 = norm.reshape(-1, C)
    partial = _sc_call(emb, gidx2, dst2, norm2)
    return _sum_partials(partial)


# speedup vs baseline: 6.8447x; 1.4125x over previous
import functools

import jax
import jax.numpy as jnp
from jax import lax
from jax.experimental import pallas as pl
from jax.experimental.pallas import tpu as pltpu
from jax.experimental.pallas import tpu_sc as plsc

NUM_NODES = 10000
N_EDGES = 320000
IN_FEAT = 10000
OUT_FEAT = 128
NUM_RELS = 8
NUM_BASES = 4

NC, NS, L = 2, 16, 16
NW = NC * NS
EPW = N_EDGES // NW
C = 80
NCH = EPW // C
RPT = NUM_NODES // NS
FV = OUT_FEAT // L


_QB = 400


def _embed_body(wc_ref, w3_ref, out_ref):
    for r in range(NUM_RELS):
        acc = wc_ref[r, 0] * w3_ref[:, 0, :]
        for b in range(1, NUM_BASES):
            acc = acc + wc_ref[r, b] * w3_ref[:, b, :]
        out_ref[r, :, :] = acc


def _build_embed(w_comp, w3):
    return pl.pallas_call(
        _embed_body,
        grid=(IN_FEAT // _QB,),
        in_specs=[
            pl.BlockSpec(memory_space=pltpu.SMEM),
            pl.BlockSpec((_QB, NUM_BASES, OUT_FEAT), lambda i: (i, 0, 0)),
        ],
        out_specs=pl.BlockSpec((NUM_RELS, _QB, OUT_FEAT), lambda i: (0, i, 0)),
        out_shape=jax.ShapeDtypeStruct((NUM_RELS, IN_FEAT, OUT_FEAT), jnp.float32),
    )(w_comp, w3)


def _gidx_body(src_ref, rel_ref, o_ref):
    j = rel_ref[...] * IN_FEAT + src_ref[...]
    o_ref[...] = (j & 7) * IN_FEAT + (j >> 3)


def _build_gidx(src2d, rel2d):
    return pl.pallas_call(
        _gidx_body,
        out_shape=jax.ShapeDtypeStruct(src2d.shape, jnp.int32),
    )(src2d, rel2d)


def _sc_body(emb, gidx2, dst2, norm2, out,
             idx_v, dst_v, norm_v, rows0, rows1, h_sh, g0, g1, s0, s1):
    c = lax.axis_index("c")
    s = lax.axis_index("s")
    wid = s * NC + c
    base = wid * NCH
    rows = (rows0, rows1)
    gsem = (g0, g1)
    ssem = (s0, s1)

    pltpu.sync_copy(gidx2.at[pl.ds(base, NCH)], idx_v)
    pltpu.sync_copy(dst2.at[pl.ds(base, NCH)], dst_v)
    pltpu.sync_copy(norm2.at[pl.ds(base, NCH)], norm_v)

    zero = jnp.zeros((L,), jnp.float32)

    def _zb(i, _):
        for k in range(FV):
            rows0[i, pl.ds(k * L, L)] = zero
        return 0

    lax.fori_loop(0, C, _zb, 0)
    for t in range(8):
        off = s * RPT + t * C
        n = C if t < 7 else RPT - 7 * C
        pltpu.sync_copy(rows0.at[pl.ds(0, n)], h_sh.at[pl.ds(off, n)])
    plsc.subcore_barrier()

    def _gather(j, b):
        return pltpu.async_copy(emb.at[idx_v.at[j]], rows[b], gsem[b])

    def _scale(j, b):
        rb = rows[b]

        def _sg(g, _):
            nvec = norm_v[j, pl.ds(g * L, L)]
            for t in range(L):
                nv = nvec[t]
                i = g * L + t
                for k in range(FV):
                    sl = pl.ds(k * L, L)
                    rb[i, sl] = rb[i, sl] * nv
            return 0

        lax.fori_loop(0, C // L, _sg, 0)

    def _scatter(j, b):
        return pltpu.async_copy(rows[b], h_sh.at[dst_v.at[j]], ssem[b], add=True)

    def _drain_g(b):
        pltpu.make_async_copy(emb.at[idx_v.at[0]], rows[b], gsem[b]).wait()

    def _drain_s(b):
        pltpu.make_async_copy(rows[b], h_sh.at[dst_v.at[0]], ssem[b]).wait()

    _gather(0, 0)
    _gather(1, 1)

    def _turn(j, b):
        _drain_g(b)
        _scale(j, b)
        _scatter(j, b)
        _drain_s(b)
        _gather(j + 2, b)

    def _main(jj, _):
        _turn(2 * jj, 0)
        _turn(2 * jj + 1, 1)
        return 0

    lax.fori_loop(0, (NCH - 3) // 2, _main, 0)
    _drain_g(0)
    _scale(NCH - 3, 0)
    _scatter(NCH - 3, 0)
    _drain_s(0)
    _gather(NCH - 1, 0)
    _drain_g(1)
    _scale(NCH - 2, 1)
    _scatter(NCH - 2, 1)
    _drain_s(1)
    _drain_g(0)
    _scale(NCH - 1, 0)
    _scatter(NCH - 1, 0)
    _drain_s(0)
    plsc.subcore_barrier()

    for t in range(8):
        off = s * RPT + t * C
        n = C if t < 7 else RPT - 7 * C
        pltpu.sync_copy(h_sh.at[pl.ds(off, n)], rows0.at[pl.ds(0, n)])
        pltpu.sync_copy(rows0.at[pl.ds(0, n)], out.at[c, pl.ds(off, n)])


def _sc_call(emb, gidx2, dst2, norm2):
    mesh = plsc.VectorSubcoreMesh(core_axis_name="c", subcore_axis_name="s")
    f = pl.kernel(
        _sc_body,
        out_type=jax.ShapeDtypeStruct((NC, NUM_NODES, OUT_FEAT), jnp.float32),
        mesh=mesh,
        compiler_params=pltpu.CompilerParams(use_tc_tiling_on_sc=False),
        scratch_types=[
            pltpu.VMEM((NCH, C), jnp.int32),
            pltpu.VMEM((NCH, C), jnp.int32),
            pltpu.VMEM((NCH, C), jnp.float32),
            pltpu.VMEM((C, OUT_FEAT), jnp.float32),
            pltpu.VMEM((C, OUT_FEAT), jnp.float32),
            pltpu.VMEM_SHARED((NUM_NODES, OUT_FEAT), jnp.float32),
            pltpu.SemaphoreType.DMA,
            pltpu.SemaphoreType.DMA,
            pltpu.SemaphoreType.DMA,
            pltpu.SemaphoreType.DMA,
        ],
    )
    return f(emb, gidx2, dst2, norm2)


_RB = 2000


def _sum_body(p_ref, o_ref):
    o_ref[...] = p_ref[0] + p_ref[1]


def _sum_partials(partial):
    return pl.pallas_call(
        _sum_body,
        grid=(NUM_NODES // _RB,),
        in_specs=[pl.BlockSpec((NC, _RB, OUT_FEAT), lambda i: (0, i, 0))],
        out_specs=pl.BlockSpec((_RB, OUT_FEAT), lambda i: (i, 0)),
        out_shape=jax.ShapeDtypeStruct((NUM_NODES, OUT_FEAT), jnp.float32),
    )(partial)


def kernel(src_id, dst_id, rel_type, norm, weight, w_comp):
    w3 = weight.reshape(IN_FEAT, NUM_BASES, OUT_FEAT)
    emb = _build_embed(w_comp, w3).reshape(NUM_RELS * IN_FEAT, OUT_FEAT)
    gidx = _build_gidx(src_id.reshape(-1, OUT_FEAT), rel_type.reshape(-1, OUT_FEAT))
    gidx2 = gidx.reshape(-1, C)
    dst2 = dst_id.reshape(-1, C)
    norm2 = norm.reshape(-1, C)
    partial = _sc_call(emb, gidx2, dst2, norm2)
    return _sum_partials(partial)
